# baseline (speedup 1.0000x reference)
import functools

import jax
import jax.numpy as jnp
from jax import lax
from jax.experimental import pallas as pl
from jax.experimental.pallas import tpu as pltpu
from jax.experimental.pallas import tpu_sc as plsc

D_MODEL = 128
NUM_CORES = 2
NUM_SUBCORES = 16
NUM_WORKERS = NUM_CORES * NUM_SUBCORES
CHUNK = 128
NBUF = 4


@jax.jit
def _run(x, idx2d, pe2d):
    n = x.shape[0]
    rows_per_w = n // NUM_WORKERS
    chunks = rows_per_w // CHUNK
    mesh = plsc.VectorSubcoreMesh(core_axis_name="c", subcore_axis_name="s")

    @functools.partial(
        pl.kernel,
        mesh=mesh,
        out_type=jax.ShapeDtypeStruct((n, D_MODEL), jnp.float32),
        scratch_types=[
            pltpu.VMEM_SHARED((100, D_MODEL), jnp.float32),
            pltpu.VMEM((chunks, CHUNK), jnp.int32),
        ]
        + [pltpu.VMEM((CHUNK, D_MODEL), jnp.float32) for _ in range(NBUF)]
        + [pltpu.SemaphoreType.DMA for _ in range(3 * NBUF)],
    )
    def k(x_hbm, idx_hbm, pe_hbm, out_hbm, pe_sh, idxs, *rest):
        xb = rest[:NBUF]
        sin = rest[NBUF:2 * NBUF]
        sadd = rest[2 * NBUF:3 * NBUF]
        sout = rest[3 * NBUF:4 * NBUF]
        wid = lax.axis_index("s") * NUM_CORES + lax.axis_index("c")

        @pl.when(lax.axis_index("s") == 0)
        def _():
            pltpu.sync_copy(pe_hbm, pe_sh)

        pltpu.sync_copy(idx_hbm.at[pl.ds(wid * chunks, chunks)], idxs)
        plsc.subcore_barrier()

        def in_copy(g, b):
            base = wid * rows_per_w + g * CHUNK
            return pltpu.make_async_copy(
                x_hbm.at[pl.ds(base, CHUNK)], xb[b], sin[b])

        def add_copy(g, b):
            return pltpu.make_async_copy(pe_sh.at[idxs.at[g]], xb[b], sadd[b])

        def out_copy(g, b):
            base = wid * rows_per_w + g * CHUNK
            return pltpu.make_async_copy(
                xb[b], out_hbm.at[pl.ds(base, CHUNK)], sout[b])

        in_copy(0, 0).start()
        in_copy(1, 1).start()

        def step(g, b):
            b2 = (b + 2) % NBUF
            b1 = (b - 1) % NBUF

            @pl.when(g + 2 < chunks)
            def _():
                @pl.when(g >= 2)
                def _():
                    out_copy(g - 2, b2).wait()

                in_copy(g + 2, b2).start()

            in_copy(g, b).wait()
            add_copy(g, b).start(add=True)

            @pl.when(g >= 1)
            def _():
                add_copy(g - 1, b1).wait()
                out_copy(g - 1, b1).start()

        def quad(h, carry):
            for u in range(NBUF):
                step(NBUF * h + u, u)
            return carry

        lax.fori_loop(0, chunks // NBUF, quad, 0)
        last = chunks - 1
        add_copy(last, last % NBUF).wait()
        out_copy(last, last % NBUF).start()
        for g in range(chunks - 4, chunks):
            out_copy(g, g % NBUF).wait()

    return k(x, idx2d, pe2d)


def kernel(x, layer_index, pe):
    pe2d = pe.reshape(pe.shape[0], pe.shape[-1])
    idx2d = layer_index.reshape(layer_index.shape[0] // CHUNK, CHUNK)
    return _run(x, idx2d, pe2d)
